# baseline (speedup 1.0000x reference)
import functools

import jax
import jax.numpy as jnp
from jax import lax
from jax.experimental import pallas as pl
from jax.experimental.pallas import tpu as pltpu
from jax.experimental.pallas import tpu_sc as plsc

K_NN = 16
_EPS = 1e-6


def _knn_body(n_total, q_ref, ptsT_ref, idx_ref):
    q = q_ref[0]
    ptsT = ptsT_ref[0]
    R = q.shape[0]
    N = ptsT.shape[1]
    qsq = jnp.sum(q * q, axis=1, keepdims=True)
    psq = jnp.sum(ptsT * ptsT, axis=0, keepdims=True)
    qp = lax.dot_general(q, ptsT, (((1,), (0,)), ((), ())),
                         preferred_element_type=jnp.float32)
    d = qsq + psq - 2.0 * qp

    C = 128
    NCH = N // C
    S = K_NN
    d3 = d.reshape(R, C, NCH)
    cmin = jnp.min(d3, axis=1)
    ch_iota = lax.broadcasted_iota(jnp.int32, (R, NCH), 1)
    cm = cmin
    ct = []
    for _ in range(S):
        c_t = jnp.argmin(cm, axis=1).astype(jnp.int32)[:, None]
        ct.append(c_t)
        cm = jnp.where(ch_iota == c_t, jnp.inf, cm)
    ct_sel = jnp.concatenate(ct, axis=1)
    gidx = jnp.broadcast_to(ct_sel[:, None, :], (R, C, S))
    cand = jnp.take_along_axis(d3, gidx, axis=2).reshape(R, C * S)

    iota2 = lax.broadcasted_iota(jnp.int32, (R, C * S), 1)
    siota = lax.broadcasted_iota(jnp.int32, (R, S), 1)
    cols = []
    for _ in range(K_NN):
        p = jnp.argmin(cand, axis=1).astype(jnp.int32)[:, None]
        slot = p % S
        cid = jnp.min(jnp.where(siota == slot, ct_sel, NCH), axis=1,
                      keepdims=True)
        cols.append(cid * C + p // S)
        cand = jnp.where(iota2 == p, jnp.inf, cand)
    idx = jnp.concatenate(cols, axis=1)
    idx_ref[0] = idx + pl.program_id(0) * n_total


def _knn_indices(coords):
    B, N, _ = coords.shape
    R = 512
    C = 128
    perm = (jnp.arange(N // C)[None, :] * C
            + jnp.arange(C)[:, None]).reshape(-1)
    coordsT = jnp.transpose(coords, (0, 2, 1))[:, :, perm]
    return pl.pallas_call(
        functools.partial(_knn_body, N),
        grid=(B, N // R),
        in_specs=[
            pl.BlockSpec((1, R, 3), lambda b, i: (b, i, 0)),
            pl.BlockSpec((1, 3, N), lambda b, i: (b, 0, 0)),
        ],
        out_specs=pl.BlockSpec((1, R, K_NN), lambda b, i: (b, i, 0)),
        out_shape=jax.ShapeDtypeStruct((B, N, K_NN), jnp.int32),
        compiler_params=pltpu.CompilerParams(
            dimension_semantics=("parallel", "parallel")),
    )(coords, coordsT)


_GD = 16
_GC = 1024


def _sc_gather(table, idx_flat):
    info = plsc.get_sparse_core_info()
    nc, ns = info.num_cores, info.num_subcores
    nw = nc * ns
    bi = idx_flat.shape[0]
    assert bi % (8 * nw) == 0
    b_per_w = bi // nw
    chunk = min(_GC, b_per_w)
    mesh = plsc.VectorSubcoreMesh(core_axis_name="c", subcore_axis_name="s")

    @functools.partial(
        pl.kernel, mesh=mesh,
        out_type=jax.ShapeDtypeStruct((bi, _GD), jnp.float32),
        compiler_params=pltpu.CompilerParams(use_tc_tiling_on_sc=False),
        scratch_types=[
            pltpu.VMEM((chunk,), jnp.int32),
            pltpu.VMEM((chunk, _GD), jnp.float32),
            pltpu.SemaphoreType.DMA,
        ],
    )
    def gk(table_hbm, idx_hbm, out_hbm, idx_v, rows_v, sem):
        wid = lax.axis_index("s") * nc + lax.axis_index("c")
        base = wid * b_per_w
        for c in range(b_per_w // chunk):
            off = base + c * chunk
            pltpu.sync_copy(idx_hbm.at[pl.ds(off, chunk)], idx_v)
            pltpu.async_copy(table_hbm.at[idx_v], rows_v, sem).wait()
            pltpu.sync_copy(rows_v, out_hbm.at[pl.ds(off, chunk)])

    return gk(table, idx_flat)


def _leaky(x, s):
    return jnp.where(x >= 0, x, s * x)


def _pipe_body(c_ref, nbr_ref, f_ref,
               W1T_ref, b1_ref, Wl12T_ref, bl12_ref,
               A1_ref, B1_ref, Wp1t_ref, Wp1b_ref, bp1_ref,
               A2_ref, B2_ref, Wp2t_ref, Wp2b_ref, bp2_ref,
               W2T_ref, b2_ref, WscT_ref, bsc_ref,
               out_ref):
    ext = c_ref[0]
    feat = f_ref[0]
    P = ext.shape[0]

    def mm(x, wT_ref, b_ref=None):
        y = lax.dot_general(x, wT_ref[...], (((1,), (0,)), ((), ())),
                            preferred_element_type=jnp.float32)
        if b_ref is not None:
            y = y + b_ref[...]
        return y

    x1 = _leaky(mm(feat, W1T_ref, b1_ref), 0.2)
    sc = mm(feat, WscT_ref, bsc_ref)

    nbr_all = jnp.concatenate([nbr_ref[0, k] for k in range(K_NN)], axis=0)
    ext_all = jnp.concatenate([ext] * K_NN, axis=0)
    ones = jnp.ones((K_NN * P, 1), jnp.float32)
    geo = jnp.concatenate([ext_all, nbr_all, ext_all - nbr_all, ones],
                          axis=1)
    y12 = jnp.maximum(mm(geo, Wl12T_ref, bl12_ref), 0.0)

    def pool(y, pf, A_ref, B_ref, Wpt_ref, Wpb_ref, bp_ref):
        s_pf = mm(pf, B_ref)
        sk = [mm(y[k * P:(k + 1) * P], A_ref) + s_pf for k in range(K_NN)]
        m = sk[0]
        for k in range(1, K_NN):
            m = jnp.maximum(m, sk[k])
        acc = None
        z = None
        for k in range(K_NN):
            e = jnp.exp(sk[k] - m)
            w = e * y[k * P:(k + 1) * P]
            acc = w if acc is None else acc + w
            z = e if z is None else z + e
        agg = acc / z
        return jnp.maximum(mm(agg, Wpt_ref) + mm(pf, Wpb_ref) + bp_ref[...],
                           0.0)

    p1 = pool(y12[:, :32], x1, A1_ref, B1_ref, Wp1t_ref, Wp1b_ref, bp1_ref)
    p2 = pool(y12[:, 32:], p1, A2_ref, B2_ref, Wp2t_ref, Wp2b_ref, bp2_ref)

    out_ref[0] = _leaky(mm(p2, W2T_ref, b2_ref) + sc, 0.01)


def _pipeline(coords, nbrK, featN, weights):
    B, N, _ = coords.shape
    P = 512
    w_specs = [pl.BlockSpec(w.shape, lambda b, i: (0,) * w.ndim)
               for w in weights]
    return pl.pallas_call(
        _pipe_body,
        grid=(B, N // P),
        in_specs=[
            pl.BlockSpec((1, P, 3), lambda b, i: (b, i, 0)),
            pl.BlockSpec((1, K_NN, P, 3), lambda b, i: (b, 0, i, 0)),
            pl.BlockSpec((1, P, 32), lambda b, i: (b, i, 0)),
        ] + w_specs,
        out_specs=pl.BlockSpec((1, P, 128), lambda b, i: (b, i, 0)),
        out_shape=jax.ShapeDtypeStruct((B, N, 128), jnp.float32),
        compiler_params=pltpu.CompilerParams(
            dimension_semantics=("parallel", "parallel")),
    )(coords, nbrK, featN, *weights)


def kernel(coords, features, W1, b1, Wl1, bl1, gl1, bel1, Ws1, Wp1, bp1,
           gp1, bep1, Wl2, bl2, gl2, bel2, Ws2, Wp2, bp2, gp2, bep2,
           W2, b2, Wsc, bsc, gsc, besc):
    B, N, _ = coords.shape
    idx = _knn_indices(coords)

    table = jnp.pad(coords.reshape(B * N, 3), ((0, 0), (0, _GD - 3)))
    rows = _sc_gather(table, idx.reshape(-1))
    nbr = rows[:, :3].reshape(B, N, K_NN, 3)
    nbrK = jnp.transpose(nbr, (0, 2, 1, 3))

    featN = jnp.transpose(features[:, :, :, 0], (0, 2, 1))

    inv = 1.0 / jnp.sqrt(1.0 + _EPS)

    def fold(W, b, g, be):
        s = g * inv
        return (W * s[:, None]).T, (b * s + be)[None, :]

    Wl1T, bl1f = fold(Wl1, bl1, gl1, bel1)
    Wp1T, bp1f = fold(Wp1, bp1, gp1, bep1)
    Wl2T, bl2f = fold(Wl2, bl2, gl2, bel2)
    Wp2T, bp2f = fold(Wp2, bp2, gp2, bep2)
    WscT, bscf = fold(Wsc, bsc, gsc, besc)
    Ws1T, Ws2T = Ws1.T, Ws2.T
    weights = [W1.T, b1[None, :],
               jnp.concatenate([Wl1T, Wl2T], axis=1),
               jnp.concatenate([bl1f, bl2f], axis=1),
               Ws1T[:32, :32], Ws1T[32:, :32], Wp1T[:32], Wp1T[32:], bp1f,
               Ws2T[:32, :32], Ws2T[32:, :32], Wp2T[:32], Wp2T[32:], bp2f,
               W2.T, b2[None, :], WscT, bscf]

    out = _pipeline(coords, nbrK, featN, weights)
    return jnp.transpose(out, (0, 2, 1))[:, :, :, None]


# speedup vs baseline: 1.0162x; 1.0162x over previous
import functools

import jax
import jax.numpy as jnp
from jax import lax
from jax.experimental import pallas as pl
from jax.experimental.pallas import tpu as pltpu
from jax.experimental.pallas import tpu_sc as plsc

K_NN = 16
_EPS = 1e-6


def _knn_body(n_total, q_ref, ptsT_ref, idx_ref):
    q = q_ref[0]
    ptsT = ptsT_ref[0]
    R = q.shape[0]
    N = ptsT.shape[1]
    qsq = jnp.sum(q * q, axis=1, keepdims=True)
    psq = jnp.sum(ptsT * ptsT, axis=0, keepdims=True)
    qp = lax.dot_general(q, ptsT, (((1,), (0,)), ((), ())),
                         preferred_element_type=jnp.float32)
    d = qsq + psq - 2.0 * qp
    iota = lax.broadcasted_iota(jnp.int32, (R, N), 1)
    cols = []
    for t in range(K_NN):
        am = jnp.argmin(d, axis=1).astype(jnp.int32)[:, None]
        cols.append(am)
        if t < K_NN - 1:
            d = jnp.where(iota == am, jnp.inf, d)
    idx = jnp.concatenate(cols, axis=1)
    idx_ref[0] = idx + pl.program_id(0) * n_total


def _knn_indices(coords):
    B, N, _ = coords.shape
    R = 512
    coordsT = jnp.transpose(coords, (0, 2, 1))
    return pl.pallas_call(
        functools.partial(_knn_body, N),
        grid=(B, N // R),
        in_specs=[
            pl.BlockSpec((1, R, 3), lambda b, i: (b, i, 0)),
            pl.BlockSpec((1, 3, N), lambda b, i: (b, 0, 0)),
        ],
        out_specs=pl.BlockSpec((1, R, K_NN), lambda b, i: (b, i, 0)),
        out_shape=jax.ShapeDtypeStruct((B, N, K_NN), jnp.int32),
        compiler_params=pltpu.CompilerParams(
            dimension_semantics=("parallel", "parallel")),
    )(coords, coordsT)


_GD = 16
_GC = 1024


def _sc_gather(table, idx_flat):
    info = plsc.get_sparse_core_info()
    nc, ns = info.num_cores, info.num_subcores
    nw = nc * ns
    bi = idx_flat.shape[0]
    assert bi % (8 * nw) == 0
    b_per_w = bi // nw
    chunk = min(_GC, b_per_w)
    mesh = plsc.VectorSubcoreMesh(core_axis_name="c", subcore_axis_name="s")

    @functools.partial(
        pl.kernel, mesh=mesh,
        out_type=jax.ShapeDtypeStruct((bi, _GD), jnp.float32),
        compiler_params=pltpu.CompilerParams(use_tc_tiling_on_sc=False),
        scratch_types=[
            pltpu.VMEM((chunk,), jnp.int32),
            pltpu.VMEM((chunk, _GD), jnp.float32),
            pltpu.SemaphoreType.DMA,
        ],
    )
    def gk(table_hbm, idx_hbm, out_hbm, idx_v, rows_v, sem):
        wid = lax.axis_index("s") * nc + lax.axis_index("c")
        base = wid * b_per_w
        for c in range(b_per_w // chunk):
            off = base + c * chunk
            pltpu.sync_copy(idx_hbm.at[pl.ds(off, chunk)], idx_v)
            pltpu.async_copy(table_hbm.at[idx_v], rows_v, sem).wait()
            pltpu.sync_copy(rows_v, out_hbm.at[pl.ds(off, chunk)])

    return gk(table, idx_flat)


def _leaky(x, s):
    return jnp.where(x >= 0, x, s * x)


def _pipe_body(c_ref, nbr_ref, f_ref,
               W1T_ref, b1_ref, Wl12T_ref, bl12_ref,
               A1_ref, B1_ref, Wp1t_ref, Wp1b_ref, bp1_ref,
               A2_ref, B2_ref, Wp2t_ref, Wp2b_ref, bp2_ref,
               W2T_ref, b2_ref, WscT_ref, bsc_ref,
               out_ref):
    ext = c_ref[0]
    feat = f_ref[0]
    P = ext.shape[0]

    def mm(x, wT_ref, b_ref=None):
        y = lax.dot_general(x, wT_ref[...], (((1,), (0,)), ((), ())),
                            preferred_element_type=jnp.float32)
        if b_ref is not None:
            y = y + b_ref[...]
        return y

    x1 = _leaky(mm(feat, W1T_ref, b1_ref), 0.2)
    sc = mm(feat, WscT_ref, bsc_ref)

    nbr_all = jnp.concatenate([nbr_ref[0, k] for k in range(K_NN)], axis=0)
    ext_all = jnp.concatenate([ext] * K_NN, axis=0)
    ones = jnp.ones((K_NN * P, 1), jnp.float32)
    geo = jnp.concatenate([ext_all, nbr_all, ext_all - nbr_all, ones],
                          axis=1)
    y12 = jnp.maximum(mm(geo, Wl12T_ref, bl12_ref), 0.0)

    def pool(y, pf, A_ref, B_ref, Wpt_ref, Wpb_ref, bp_ref):
        s_pf = mm(pf, B_ref)
        sk = [mm(y[k * P:(k + 1) * P], A_ref) + s_pf for k in range(K_NN)]
        m = sk[0]
        for k in range(1, K_NN):
            m = jnp.maximum(m, sk[k])
        acc = None
        z = None
        for k in range(K_NN):
            e = jnp.exp(sk[k] - m)
            w = e * y[k * P:(k + 1) * P]
            acc = w if acc is None else acc + w
            z = e if z is None else z + e
        agg = acc / z
        return jnp.maximum(mm(agg, Wpt_ref) + mm(pf, Wpb_ref) + bp_ref[...],
                           0.0)

    p1 = pool(y12[:, :32], x1, A1_ref, B1_ref, Wp1t_ref, Wp1b_ref, bp1_ref)
    p2 = pool(y12[:, 32:], p1, A2_ref, B2_ref, Wp2t_ref, Wp2b_ref, bp2_ref)

    out_ref[0] = _leaky(mm(p2, W2T_ref, b2_ref) + sc, 0.01)


def _pipeline(coords, nbrK, featN, weights):
    B, N, _ = coords.shape
    P = 512
    w_specs = [pl.BlockSpec(w.shape, lambda b, i: (0,) * w.ndim)
               for w in weights]
    return pl.pallas_call(
        _pipe_body,
        grid=(B, N // P),
        in_specs=[
            pl.BlockSpec((1, P, 3), lambda b, i: (b, i, 0)),
            pl.BlockSpec((1, K_NN, P, 3), lambda b, i: (b, 0, i, 0)),
            pl.BlockSpec((1, P, 32), lambda b, i: (b, i, 0)),
        ] + w_specs,
        out_specs=pl.BlockSpec((1, P, 128), lambda b, i: (b, i, 0)),
        out_shape=jax.ShapeDtypeStruct((B, N, 128), jnp.float32),
        compiler_params=pltpu.CompilerParams(
            dimension_semantics=("parallel", "parallel")),
    )(coords, nbrK, featN, *weights)


def kernel(coords, features, W1, b1, Wl1, bl1, gl1, bel1, Ws1, Wp1, bp1,
           gp1, bep1, Wl2, bl2, gl2, bel2, Ws2, Wp2, bp2, gp2, bep2,
           W2, b2, Wsc, bsc, gsc, besc):
    B, N, _ = coords.shape
    idx = _knn_indices(coords)

    table = jnp.pad(coords.reshape(B * N, 3), ((0, 0), (0, _GD - 3)))
    rows = _sc_gather(table, idx.reshape(-1))
    nbr = rows[:, :3].reshape(B, N, K_NN, 3)
    nbrK = jnp.transpose(nbr, (0, 2, 1, 3))

    featN = jnp.transpose(features[:, :, :, 0], (0, 2, 1))

    inv = 1.0 / jnp.sqrt(1.0 + _EPS)

    def fold(W, b, g, be):
        s = g * inv
        return (W * s[:, None]).T, (b * s + be)[None, :]

    Wl1T, bl1f = fold(Wl1, bl1, gl1, bel1)
    Wp1T, bp1f = fold(Wp1, bp1, gp1, bep1)
    Wl2T, bl2f = fold(Wl2, bl2, gl2, bel2)
    Wp2T, bp2f = fold(Wp2, bp2, gp2, bep2)
    WscT, bscf = fold(Wsc, bsc, gsc, besc)
    Ws1T, Ws2T = Ws1.T, Ws2.T
    weights = [W1.T, b1[None, :],
               jnp.concatenate([Wl1T, Wl2T], axis=1),
               jnp.concatenate([bl1f, bl2f], axis=1),
               Ws1T[:32, :32], Ws1T[32:, :32], Wp1T[:32], Wp1T[32:], bp1f,
               Ws2T[:32, :32], Ws2T[32:, :32], Wp2T[:32], Wp2T[32:], bp2f,
               W2.T, b2[None, :], WscT, bscf]

    out = _pipeline(coords, nbrK, featN, weights)
    return jnp.transpose(out, (0, 2, 1))[:, :, :, None]
